# baseline (speedup 1.0000x reference)
import functools

import jax
import jax.numpy as jnp
from jax import lax
from jax.experimental import pallas as pl
from jax.experimental.pallas import tpu as pltpu
from jax.experimental.pallas import tpu_sc as plsc

_N = 10000
_NP = 10240
_E = 320000
_D = 128
_NG = 16
_DOUT = 64

_NC_CORES = 2
_NSUB = 16
_NW = _NC_CORES * _NSUB
_PER_W = _E // _NW
_CK = 80
_NCH = _PER_W // _CK
_RPT = _NP // _NSUB
_CW = 16

_mesh = plsc.VectorSubcoreMesh(core_axis_name="c", subcore_axis_name="s")


def _make_scatter_pass(gdim, sdim):
    return functools.partial(
        pl.kernel,
        out_type=jax.ShapeDtypeStruct((_NC_CORES, _NP, _D), jnp.float32),
        mesh=_mesh,
        compiler_params=pltpu.CompilerParams(use_tc_tiling_on_sc=False),
        scratch_types=[
            pltpu.VMEM((_NCH, _CK), jnp.int32),
            pltpu.VMEM((_NCH, _CK), jnp.int32),
            pltpu.VMEM((_CK, _D), jnp.float32),
            pltpu.VMEM((_CK, _D), jnp.float32),
            pltpu.SemaphoreType.DMA,
            pltpu.SemaphoreType.DMA,
            pltpu.VMEM_SHARED((_NP, _D), jnp.float32),
        ],
    )(functools.partial(_sc_scatter_pass, gdim=gdim, sdim=sdim))


def _sc_scatter_pass(src, hidx, out, gidx_v, sidx_v, rows_a, rows_b,
                     sem_a, sem_b, acc, *, gdim, sdim):
    cid = lax.axis_index("c")
    sid = lax.axis_index("s")
    wid = sid * _NC_CORES + cid

    pltpu.sync_copy(hidx.at[gdim, wid], gidx_v)
    pltpu.sync_copy(hidx.at[sdim, wid], sidx_v)

    @pl.loop(0, _CK)
    def _zero_rows(i):
        for k in range(_D // 16):
            rows_a[i, pl.ds(k * 16, 16)] = jnp.zeros((16,), jnp.float32)

    for t in range(_RPT // _CK):
        pltpu.sync_copy(rows_a, acc.at[pl.ds(sid * _RPT + t * _CK, _CK)])
    plsc.subcore_barrier()

    pltpu.async_copy(src.at[gidx_v.at[0]], rows_a, sem_a)

    @pl.loop(0, _NCH - 1, step=2)
    def _chunk(j):
        pltpu.async_copy(src.at[gidx_v.at[j + 1]], rows_b, sem_b)
        pltpu.make_async_copy(src.at[gidx_v.at[j]], rows_a, sem_a).wait()
        pltpu.sync_copy(rows_a, acc.at[sidx_v.at[j]], add=True)
        pltpu.async_copy(src.at[gidx_v.at[j + 2]], rows_a, sem_a)
        pltpu.make_async_copy(src.at[gidx_v.at[j + 1]], rows_b, sem_b).wait()
        pltpu.sync_copy(rows_b, acc.at[sidx_v.at[j + 1]], add=True)

    pltpu.make_async_copy(src.at[gidx_v.at[_NCH - 1]], rows_a, sem_a).wait()
    pltpu.sync_copy(rows_a, acc.at[sidx_v.at[_NCH - 1]], add=True)

    plsc.subcore_barrier()

    nslc = _RPT // _CK
    for t in range(nslc):
        buf, sem = (rows_a, sem_a) if t % 2 == 0 else (rows_b, sem_b)
        r0 = sid * _RPT + t * _CK
        if t >= 2:
            rp = sid * _RPT + (t - 2) * _CK
            pltpu.make_async_copy(buf, out.at[cid, pl.ds(rp, _CK)],
                                  sem).wait()
        pltpu.sync_copy(acc.at[pl.ds(r0, _CK)], buf)
        pltpu.async_copy(buf, out.at[cid, pl.ds(r0, _CK)], sem)
    for t in (nslc - 2, nslc - 1):
        buf, sem = (rows_a, sem_a) if t % 2 == 0 else (rows_b, sem_b)
        r0 = sid * _RPT + t * _CK
        pltpu.make_async_copy(buf, out.at[cid, pl.ds(r0, _CK)], sem).wait()


_scatter_ne = _make_scatter_pass(0, 1)
_scatter_en = _make_scatter_pass(1, 0)


@functools.partial(
    pl.kernel,
    out_type=(jax.ShapeDtypeStruct((_NC_CORES, _NP, _D), jnp.float32),
              jax.ShapeDtypeStruct((_NC_CORES, 2, _NP, _CW), jnp.float32)),
    mesh=_mesh,
    compiler_params=pltpu.CompilerParams(use_tc_tiling_on_sc=False),
    scratch_types=[
        pltpu.VMEM((2, _CK), jnp.int32),
        pltpu.VMEM((2, _CK), jnp.int32),
        pltpu.VMEM((_CK, _D), jnp.float32),
        pltpu.VMEM((_CK, _D), jnp.float32),
        pltpu.VMEM((_CK, _CW), jnp.float32),
        pltpu.VMEM((_CK, _CW), jnp.float32),
        pltpu.SemaphoreType.DMA,
        pltpu.SemaphoreType.DMA,
        pltpu.VMEM_SHARED((_NP, _D), jnp.float32),
        pltpu.VMEM_SHARED((_NP, _CW), jnp.float32),
        pltpu.VMEM_SHARED((_NP, _CW), jnp.float32),
    ],
)
def _sc_pass1_counts(src, hidx, out, outc, ni_v, ei_v, rows_a, rows_b,
                     ones_v, z16, sem_a, sem_b, acc, acc_d, acc_b):
    cid = lax.axis_index("c")
    sid = lax.axis_index("s")
    wid = sid * _NC_CORES + cid

    @pl.loop(0, _CK)
    def _init_bufs(i):
        for k in range(_D // 16):
            rows_a[i, pl.ds(k * 16, 16)] = jnp.zeros((16,), jnp.float32)
        z16[i, :] = jnp.zeros((_CW,), jnp.float32)
        ones_v[i, :] = jnp.ones((_CW,), jnp.float32)

    for t in range(_RPT // _CK):
        r0 = sid * _RPT + t * _CK
        pltpu.sync_copy(rows_a, acc.at[pl.ds(r0, _CK)])
        pltpu.sync_copy(z16, acc_d.at[pl.ds(r0, _CK)])
        pltpu.sync_copy(z16, acc_b.at[pl.ds(r0, _CK)])
    plsc.subcore_barrier()

    pltpu.sync_copy(hidx.at[0, wid, 0], ni_v.at[0])
    pltpu.sync_copy(hidx.at[1, wid, 0], ei_v.at[0])
    pltpu.async_copy(src.at[ni_v.at[0]], rows_a, sem_a)
    pltpu.sync_copy(hidx.at[0, wid, 1], ni_v.at[1])
    pltpu.sync_copy(hidx.at[1, wid, 1], ei_v.at[1])
    pltpu.async_copy(src.at[ni_v.at[1]], rows_b, sem_b)

    @pl.loop(0, _NCH - 2, step=2)
    def _chunk(j):
        pltpu.make_async_copy(src.at[ni_v.at[0]], rows_a, sem_a).wait()
        pltpu.sync_copy(rows_a, acc.at[ei_v.at[0]], add=True)
        pltpu.sync_copy(ones_v, acc_d.at[ni_v.at[0]], add=True)
        pltpu.sync_copy(ones_v, acc_b.at[ei_v.at[0]], add=True)
        pltpu.sync_copy(hidx.at[0, wid, j + 2], ni_v.at[0])
        pltpu.sync_copy(hidx.at[1, wid, j + 2], ei_v.at[0])
        pltpu.async_copy(src.at[ni_v.at[0]], rows_a, sem_a)

        pltpu.make_async_copy(src.at[ni_v.at[1]], rows_b, sem_b).wait()
        pltpu.sync_copy(rows_b, acc.at[ei_v.at[1]], add=True)
        pltpu.sync_copy(ones_v, acc_d.at[ni_v.at[1]], add=True)
        pltpu.sync_copy(ones_v, acc_b.at[ei_v.at[1]], add=True)

        @pl.when(j + 3 < _NCH)
        def _prefetch_odd():
            pltpu.sync_copy(hidx.at[0, wid, j + 3], ni_v.at[1])
            pltpu.sync_copy(hidx.at[1, wid, j + 3], ei_v.at[1])
            pltpu.async_copy(src.at[ni_v.at[1]], rows_b, sem_b)

    pltpu.make_async_copy(src.at[ni_v.at[0]], rows_a, sem_a).wait()
    pltpu.sync_copy(rows_a, acc.at[ei_v.at[0]], add=True)
    pltpu.sync_copy(ones_v, acc_d.at[ni_v.at[0]], add=True)
    pltpu.sync_copy(ones_v, acc_b.at[ei_v.at[0]], add=True)

    plsc.subcore_barrier()

    nslc = _RPT // _CK
    for t in range(nslc):
        buf, sem = (rows_a, sem_a) if t % 2 == 0 else (rows_b, sem_b)
        r0 = sid * _RPT + t * _CK
        if t >= 2:
            rp = sid * _RPT + (t - 2) * _CK
            pltpu.make_async_copy(buf, out.at[cid, pl.ds(rp, _CK)],
                                  sem).wait()
        pltpu.sync_copy(acc.at[pl.ds(r0, _CK)], buf)
        pltpu.async_copy(buf, out.at[cid, pl.ds(r0, _CK)], sem)
    for t in (nslc - 2, nslc - 1):
        buf, sem = (rows_a, sem_a) if t % 2 == 0 else (rows_b, sem_b)
        r0 = sid * _RPT + t * _CK
        pltpu.make_async_copy(buf, out.at[cid, pl.ds(r0, _CK)], sem).wait()
    for t in range(nslc):
        r0 = sid * _RPT + t * _CK
        pltpu.sync_copy(acc_d.at[pl.ds(r0, _CK)], z16)
        pltpu.sync_copy(z16, outc.at[cid, 0, pl.ds(r0, _CK)])
        pltpu.sync_copy(acc_b.at[pl.ds(r0, _CK)], ones_v)
        pltpu.sync_copy(ones_v, outc.at[cid, 1, pl.ds(r0, _CK)])


_BLK = 1024
_GRID = _NP // _BLK


def _mm_body(x_ref, w_ref, o_ref):
    o_ref[...] = jnp.dot(x_ref[...], w_ref[...],
                         preferred_element_type=jnp.float32)


def _tc_matmul(x, w):
    return pl.pallas_call(
        _mm_body,
        grid=(_GRID,),
        in_specs=[pl.BlockSpec((_BLK, _D), lambda i: (i, 0)),
                  pl.BlockSpec((_D, _D), lambda i: (0, 0))],
        out_specs=pl.BlockSpec((_BLK, _D), lambda i: (i, 0)),
        out_shape=jax.ShapeDtypeStruct((_NP, _D), jnp.float32),
    )(x, w)


def _inv_from_counts(c_ref):
    cnt = c_ref[0, 0, :, 0:1] + c_ref[1, 0, :, 0:1]
    return jnp.where(cnt > 0.0, 1.0 / cnt, 0.0)


def _merge_b_body(p_ref, c_ref, o_ref):
    o_ref[...] = (p_ref[0] + p_ref[1]) * _inv_from_counts(c_ref)


def _tc_merge_b(parts, counts):
    return pl.pallas_call(
        _merge_b_body,
        grid=(_GRID,),
        in_specs=[
            pl.BlockSpec((_NC_CORES, _BLK, _D), lambda i: (0, i, 0)),
            pl.BlockSpec((_NC_CORES, 1, _BLK, _CW), lambda i: (0, 1, i, 0)),
        ],
        out_specs=pl.BlockSpec((_BLK, _D), lambda i: (i, 0)),
        out_shape=jax.ShapeDtypeStruct((_NP, _D), jnp.float32),
    )(parts, counts)


def _merge_d_mm_body(p_ref, c_ref, b_ref, w_ref, h_ref, o_ref):
    h = jnp.maximum(
        (p_ref[0] + p_ref[1]) * _inv_from_counts(c_ref) + b_ref[...], 0.0)
    h_ref[...] = h
    o_ref[...] = jnp.dot(h, w_ref[...], preferred_element_type=jnp.float32)


def _tc_merge_d_mm(parts, counts, bias, w):
    return pl.pallas_call(
        _merge_d_mm_body,
        grid=(_GRID,),
        in_specs=[
            pl.BlockSpec((_NC_CORES, _BLK, _D), lambda i: (0, i, 0)),
            pl.BlockSpec((_NC_CORES, 1, _BLK, _CW), lambda i: (0, 0, i, 0)),
            pl.BlockSpec((1, _D), lambda i: (0, 0)),
            pl.BlockSpec((_D, _D), lambda i: (0, 0)),
        ],
        out_specs=[pl.BlockSpec((_BLK, _D), lambda i: (i, 0)),
                   pl.BlockSpec((_BLK, _D), lambda i: (i, 0))],
        out_shape=[jax.ShapeDtypeStruct((_NP, _D), jnp.float32),
                   jax.ShapeDtypeStruct((_NP, _D), jnp.float32)],
    )(parts, counts, bias, w)


def _pool_body(p_ref, c_ref, b_ref, h0_ref, bt_ref, wl_ref, bl_ref, o_ref,
               sum_acc, max_acc, cnt_acc):
    i = pl.program_id(0)

    @pl.when(i == 0)
    def _():
        sum_acc[...] = jnp.zeros((_NG, _D), jnp.float32)
        cnt_acc[...] = jnp.zeros((_NG, _D), jnp.float32)
        max_acc[...] = jnp.full((_NG, _D), -jnp.inf, jnp.float32)

    m = (p_ref[0] + p_ref[1]) * _inv_from_counts(c_ref)
    h = jnp.maximum(h0_ref[...] + m + b_ref[...], 0.0)
    bt = bt_ref[...]
    oh = (bt == lax.broadcasted_iota(jnp.int32, (_BLK, _NG), 1)
          ).astype(jnp.float32)
    dn = (((0,), (0,)), ((), ()))
    sum_acc[...] += lax.dot_general(oh, h, dn,
                                    preferred_element_type=jnp.float32)
    cnt_acc[...] += lax.dot_general(oh, jnp.ones_like(h), dn,
                                    preferred_element_type=jnp.float32)
    g_lo = bt[0, 0]
    g_hi = bt[_BLK - 1, 0]
    for g in range(_NG):
        @pl.when((g_lo <= g) & (g <= g_hi))
        def _(g=g):
            mg = jnp.max(jnp.where(bt[:, 0:1] == g, h, -jnp.inf), axis=0,
                         keepdims=True)
            max_acc[g:g + 1, :] = jnp.maximum(max_acc[g:g + 1, :], mg)

    @pl.when(i == _GRID - 1)
    def _():
        s = sum_acc[...]
        mean = s / jnp.maximum(cnt_acc[...], 1.0)
        pooled = jnp.concatenate([mean, max_acc[...], s], axis=1)
        o_ref[...] = jnp.dot(pooled, wl_ref[...],
                             preferred_element_type=jnp.float32) + bl_ref[...]


def _tc_pool(parts, counts, bias, h0, bt, wlin, blin):
    return pl.pallas_call(
        _pool_body,
        grid=(_GRID,),
        in_specs=[pl.BlockSpec((_NC_CORES, _BLK, _D), lambda i: (0, i, 0)),
                  pl.BlockSpec((_NC_CORES, 1, _BLK, _CW),
                               lambda i: (0, 0, i, 0)),
                  pl.BlockSpec((1, _D), lambda i: (0, 0)),
                  pl.BlockSpec((_BLK, _D), lambda i: (i, 0)),
                  pl.BlockSpec((_BLK, _NG), lambda i: (i, 0)),
                  pl.BlockSpec((3 * _D, _DOUT), lambda i: (0, 0)),
                  pl.BlockSpec((1, _DOUT), lambda i: (0, 0))],
        out_specs=pl.BlockSpec((_NG, _DOUT), lambda i: (0, 0)),
        out_shape=jax.ShapeDtypeStruct((_NG, _DOUT), jnp.float32),
        scratch_shapes=[pltpu.VMEM((_NG, _D), jnp.float32),
                        pltpu.VMEM((_NG, _D), jnp.float32),
                        pltpu.VMEM((_NG, _D), jnp.float32)],
    )(parts, counts, bias, h0, bt, wlin, blin)


def kernel(x, hyperedge_index, edge_attr, batch, W1, b1, W2, b2, Wlin, blin):
    del edge_attr
    hi = hyperedge_index.reshape(2, _NW, _NCH, _CK)
    xp = jnp.pad(x, ((0, _NP - _N), (0, 0)))

    def half_conv(vt, counts):
        p = _scatter_ne(vt, hi)
        m = _tc_merge_b(p, counts)
        return _scatter_en(m, hi)

    p1, counts = _sc_pass1_counts(_tc_matmul(xp, W1), hi)
    q1 = _scatter_en(_tc_merge_b(p1, counts), hi)
    h, xt2 = _tc_merge_d_mm(q1, counts, b1.reshape(1, _D), W2)
    q2 = half_conv(xt2, counts)

    bt = jnp.pad(batch.astype(jnp.int32), (0, _NP - _N),
                 constant_values=_NG)
    bt = jnp.broadcast_to(bt[:, None], (_NP, _NG))
    return _tc_pool(q2, counts, b2.reshape(1, _D), h, bt, Wlin,
                    blin.reshape(1, _DOUT))


# speedup vs baseline: 1.1490x; 1.1490x over previous
import functools

import jax
import jax.numpy as jnp
from jax import lax
from jax.experimental import pallas as pl
from jax.experimental.pallas import tpu as pltpu
from jax.experimental.pallas import tpu_sc as plsc

_N = 10000
_NP = 10240
_E = 320000
_D = 128
_NG = 16
_DOUT = 64

_NC_CORES = 2
_NSUB = 16
_NW = _NC_CORES * _NSUB
_PER_W = _E // _NW
_CK = 80
_NCH = _PER_W // _CK
_RPT = _NP // _NSUB
_CW = 16

_mesh = plsc.VectorSubcoreMesh(core_axis_name="c", subcore_axis_name="s")


def _make_scatter_pass(gdim, sdim):
    return functools.partial(
        pl.kernel,
        out_type=jax.ShapeDtypeStruct((_NC_CORES, _NP, _D), jnp.float32),
        mesh=_mesh,
        compiler_params=pltpu.CompilerParams(use_tc_tiling_on_sc=False),
        scratch_types=[
            pltpu.VMEM((_NCH, _CK), jnp.int32),
            pltpu.VMEM((_NCH, _CK), jnp.int32),
            pltpu.VMEM((_CK, _D), jnp.float32),
            pltpu.VMEM((_CK, _D), jnp.float32),
            pltpu.SemaphoreType.DMA,
            pltpu.SemaphoreType.DMA,
            pltpu.VMEM_SHARED((_NP, _D), jnp.float32),
        ],
    )(functools.partial(_sc_scatter_pass, gdim=gdim, sdim=sdim))


def _sc_scatter_pass(src, hidx, out, gidx_v, sidx_v, rows_a, rows_b,
                     sem_a, sem_b, acc, *, gdim, sdim):
    cid = lax.axis_index("c")
    sid = lax.axis_index("s")
    wid = sid * _NC_CORES + cid

    pltpu.async_copy(hidx.at[gdim, wid], gidx_v, sem_a)
    pltpu.async_copy(hidx.at[sdim, wid], sidx_v, sem_b)

    @pl.loop(0, _CK)
    def _zero_rows(i):
        for k in range(_D // 16):
            rows_b[i, pl.ds(k * 16, 16)] = jnp.zeros((16,), jnp.float32)

    for t in range(_RPT // _CK):
        pltpu.sync_copy(rows_b, acc.at[pl.ds(sid * _RPT + t * _CK, _CK)])
    pltpu.make_async_copy(hidx.at[gdim, wid], gidx_v, sem_a).wait()
    pltpu.make_async_copy(hidx.at[sdim, wid], sidx_v, sem_b).wait()
    plsc.subcore_barrier()

    pltpu.async_copy(src.at[gidx_v.at[0]], rows_a, sem_a)

    @pl.loop(0, _NCH - 1, step=2)
    def _chunk(j):
        pltpu.async_copy(src.at[gidx_v.at[j + 1]], rows_b, sem_b)
        pltpu.make_async_copy(src.at[gidx_v.at[j]], rows_a, sem_a).wait()
        pltpu.sync_copy(rows_a, acc.at[sidx_v.at[j]], add=True)
        pltpu.async_copy(src.at[gidx_v.at[j + 2]], rows_a, sem_a)
        pltpu.make_async_copy(src.at[gidx_v.at[j + 1]], rows_b, sem_b).wait()
        pltpu.sync_copy(rows_b, acc.at[sidx_v.at[j + 1]], add=True)

    pltpu.make_async_copy(src.at[gidx_v.at[_NCH - 1]], rows_a, sem_a).wait()
    pltpu.sync_copy(rows_a, acc.at[sidx_v.at[_NCH - 1]], add=True)

    plsc.subcore_barrier()

    nslc = _RPT // _CK
    for t in range(nslc):
        buf, sem = (rows_a, sem_a) if t % 2 == 0 else (rows_b, sem_b)
        r0 = sid * _RPT + t * _CK
        if t >= 2:
            rp = sid * _RPT + (t - 2) * _CK
            pltpu.make_async_copy(buf, out.at[cid, pl.ds(rp, _CK)],
                                  sem).wait()
        pltpu.sync_copy(acc.at[pl.ds(r0, _CK)], buf)
        pltpu.async_copy(buf, out.at[cid, pl.ds(r0, _CK)], sem)
    for t in (nslc - 2, nslc - 1):
        buf, sem = (rows_a, sem_a) if t % 2 == 0 else (rows_b, sem_b)
        r0 = sid * _RPT + t * _CK
        pltpu.make_async_copy(buf, out.at[cid, pl.ds(r0, _CK)], sem).wait()


_scatter_ne = _make_scatter_pass(0, 1)
_scatter_en = _make_scatter_pass(1, 0)


@functools.partial(
    pl.kernel,
    out_type=jax.ShapeDtypeStruct((_NC_CORES, 2, _NP, _CW), jnp.float32),
    mesh=_mesh,
    compiler_params=pltpu.CompilerParams(use_tc_tiling_on_sc=False),
    scratch_types=[
        pltpu.VMEM((_NCH, _CK), jnp.int32),
        pltpu.VMEM((_NCH, _CK), jnp.int32),
        pltpu.VMEM((_CK, _CW), jnp.float32),
        pltpu.VMEM((_RPT, _CW), jnp.float32),
        pltpu.VMEM_SHARED((_NP, _CW), jnp.float32),
        pltpu.VMEM_SHARED((_NP, _CW), jnp.float32),
    ],
)
def _sc_counts(hidx, out, nidx_v, eidx_v, ones_v, buf_v, acc_d, acc_b):
    cid = lax.axis_index("c")
    sid = lax.axis_index("s")
    wid = sid * _NC_CORES + cid

    pltpu.sync_copy(hidx.at[0, wid], nidx_v)
    pltpu.sync_copy(hidx.at[1, wid], eidx_v)

    @pl.loop(0, _CK)
    def _fill_ones(i):
        ones_v[i, :] = jnp.ones((_CW,), jnp.float32)

    @pl.loop(0, _RPT)
    def _fill_zeros(i):
        buf_v[i, :] = jnp.zeros((_CW,), jnp.float32)

    pltpu.sync_copy(buf_v, acc_d.at[pl.ds(sid * _RPT, _RPT)])
    pltpu.sync_copy(buf_v, acc_b.at[pl.ds(sid * _RPT, _RPT)])
    plsc.subcore_barrier()

    @pl.loop(0, _NCH)
    def _chunk(j):
        pltpu.sync_copy(ones_v, acc_d.at[nidx_v.at[j]], add=True)
        pltpu.sync_copy(ones_v, acc_b.at[eidx_v.at[j]], add=True)

    plsc.subcore_barrier()

    pltpu.sync_copy(acc_d.at[pl.ds(sid * _RPT, _RPT)], buf_v)
    pltpu.sync_copy(buf_v, out.at[cid, 0, pl.ds(sid * _RPT, _RPT)])
    pltpu.sync_copy(acc_b.at[pl.ds(sid * _RPT, _RPT)], buf_v)
    pltpu.sync_copy(buf_v, out.at[cid, 1, pl.ds(sid * _RPT, _RPT)])


_BLK = 1024
_GRID = _NP // _BLK


def _mm_body(x_ref, w_ref, o_ref):
    o_ref[...] = jnp.dot(x_ref[...], w_ref[...],
                         preferred_element_type=jnp.float32)


def _tc_matmul(x, w):
    return pl.pallas_call(
        _mm_body,
        grid=(_GRID,),
        in_specs=[pl.BlockSpec((_BLK, _D), lambda i: (i, 0)),
                  pl.BlockSpec((_D, _D), lambda i: (0, 0))],
        out_specs=pl.BlockSpec((_BLK, _D), lambda i: (i, 0)),
        out_shape=jax.ShapeDtypeStruct((_NP, _D), jnp.float32),
    )(x, w)


def _inv_from_counts(c_ref):
    cnt = c_ref[0, 0, :, 0:1] + c_ref[1, 0, :, 0:1]
    return jnp.where(cnt > 0.0, 1.0 / cnt, 0.0)


def _merge_b_body(p_ref, c_ref, o_ref):
    o_ref[...] = (p_ref[0] + p_ref[1]) * _inv_from_counts(c_ref)


def _tc_merge_b(parts, counts):
    return pl.pallas_call(
        _merge_b_body,
        grid=(_GRID,),
        in_specs=[
            pl.BlockSpec((_NC_CORES, _BLK, _D), lambda i: (0, i, 0)),
            pl.BlockSpec((_NC_CORES, 1, _BLK, _CW), lambda i: (0, 1, i, 0)),
        ],
        out_specs=pl.BlockSpec((_BLK, _D), lambda i: (i, 0)),
        out_shape=jax.ShapeDtypeStruct((_NP, _D), jnp.float32),
    )(parts, counts)


def _merge_d_mm_body(p_ref, c_ref, b_ref, w_ref, h_ref, o_ref):
    h = jnp.maximum(
        (p_ref[0] + p_ref[1]) * _inv_from_counts(c_ref) + b_ref[...], 0.0)
    h_ref[...] = h
    o_ref[...] = jnp.dot(h, w_ref[...], preferred_element_type=jnp.float32)


def _tc_merge_d_mm(parts, counts, bias, w):
    return pl.pallas_call(
        _merge_d_mm_body,
        grid=(_GRID,),
        in_specs=[
            pl.BlockSpec((_NC_CORES, _BLK, _D), lambda i: (0, i, 0)),
            pl.BlockSpec((_NC_CORES, 1, _BLK, _CW), lambda i: (0, 0, i, 0)),
            pl.BlockSpec((1, _D), lambda i: (0, 0)),
            pl.BlockSpec((_D, _D), lambda i: (0, 0)),
        ],
        out_specs=[pl.BlockSpec((_BLK, _D), lambda i: (i, 0)),
                   pl.BlockSpec((_BLK, _D), lambda i: (i, 0))],
        out_shape=[jax.ShapeDtypeStruct((_NP, _D), jnp.float32),
                   jax.ShapeDtypeStruct((_NP, _D), jnp.float32)],
    )(parts, counts, bias, w)


def _pool_body(p_ref, c_ref, b_ref, h0_ref, bt_ref, wl_ref, bl_ref, o_ref,
               sum_acc, max_acc, cnt_acc):
    i = pl.program_id(0)

    @pl.when(i == 0)
    def _():
        sum_acc[...] = jnp.zeros((_NG, _D), jnp.float32)
        cnt_acc[...] = jnp.zeros((_NG, _D), jnp.float32)
        max_acc[...] = jnp.full((_NG, _D), -jnp.inf, jnp.float32)

    m = (p_ref[0] + p_ref[1]) * _inv_from_counts(c_ref)
    h = jnp.maximum(h0_ref[...] + m + b_ref[...], 0.0)
    bt = bt_ref[...]
    oh = (bt == lax.broadcasted_iota(jnp.int32, (_BLK, _NG), 1)
          ).astype(jnp.float32)
    dn = (((0,), (0,)), ((), ()))
    sum_acc[...] += lax.dot_general(oh, h, dn,
                                    preferred_element_type=jnp.float32)
    cnt_acc[...] += lax.dot_general(oh, jnp.ones_like(h), dn,
                                    preferred_element_type=jnp.float32)
    g_lo = bt[0, 0]
    g_hi = bt[_BLK - 1, 0]
    for g in range(_NG):
        @pl.when((g_lo <= g) & (g <= g_hi))
        def _(g=g):
            mg = jnp.max(jnp.where(bt[:, 0:1] == g, h, -jnp.inf), axis=0,
                         keepdims=True)
            max_acc[g:g + 1, :] = jnp.maximum(max_acc[g:g + 1, :], mg)

    @pl.when(i == _GRID - 1)
    def _():
        s = sum_acc[...]
        mean = s / jnp.maximum(cnt_acc[...], 1.0)
        pooled = jnp.concatenate([mean, max_acc[...], s], axis=1)
        o_ref[...] = jnp.dot(pooled, wl_ref[...],
                             preferred_element_type=jnp.float32) + bl_ref[...]


def _tc_pool(parts, counts, bias, h0, bt, wlin, blin):
    return pl.pallas_call(
        _pool_body,
        grid=(_GRID,),
        in_specs=[pl.BlockSpec((_NC_CORES, _BLK, _D), lambda i: (0, i, 0)),
                  pl.BlockSpec((_NC_CORES, 1, _BLK, _CW),
                               lambda i: (0, 0, i, 0)),
                  pl.BlockSpec((1, _D), lambda i: (0, 0)),
                  pl.BlockSpec((_BLK, _D), lambda i: (i, 0)),
                  pl.BlockSpec((_BLK, _NG), lambda i: (i, 0)),
                  pl.BlockSpec((3 * _D, _DOUT), lambda i: (0, 0)),
                  pl.BlockSpec((1, _DOUT), lambda i: (0, 0))],
        out_specs=pl.BlockSpec((_NG, _DOUT), lambda i: (0, 0)),
        out_shape=jax.ShapeDtypeStruct((_NG, _DOUT), jnp.float32),
        scratch_shapes=[pltpu.VMEM((_NG, _D), jnp.float32),
                        pltpu.VMEM((_NG, _D), jnp.float32),
                        pltpu.VMEM((_NG, _D), jnp.float32)],
    )(parts, counts, bias, h0, bt, wlin, blin)


def kernel(x, hyperedge_index, edge_attr, batch, W1, b1, W2, b2, Wlin, blin):
    del edge_attr
    hi = hyperedge_index.reshape(2, _NW, _NCH, _CK)
    xp = jnp.pad(x, ((0, _NP - _N), (0, 0)))

    def half_conv(vt, counts):
        p = _scatter_ne(vt, hi)
        m = _tc_merge_b(p, counts)
        return _scatter_en(m, hi)

    p1 = _scatter_ne(_tc_matmul(xp, W1), hi)
    counts = _sc_counts(hi)
    q1 = _scatter_en(_tc_merge_b(p1, counts), hi)
    h, xt2 = _tc_merge_d_mm(q1, counts, b1.reshape(1, _D), W2)
    q2 = half_conv(xt2, counts)

    bt = jnp.pad(batch.astype(jnp.int32), (0, _NP - _N),
                 constant_values=_NG)
    bt = jnp.broadcast_to(bt[:, None], (_NP, _NG))
    return _tc_pool(q2, counts, b2.reshape(1, _D), h, bt, Wlin,
                    blin.reshape(1, _DOUT))


# speedup vs baseline: 1.1575x; 1.0074x over previous
import functools

import jax
import jax.numpy as jnp
from jax import lax
from jax.experimental import pallas as pl
from jax.experimental.pallas import tpu as pltpu
from jax.experimental.pallas import tpu_sc as plsc

_N = 10000
_NP = 10240
_E = 320000
_D = 128
_NG = 16
_DOUT = 64

_NC_CORES = 2
_NSUB = 16
_NW = _NC_CORES * _NSUB
_PER_W = _E // _NW
_CK = 80
_NCH = _PER_W // _CK
_RPT = _NP // _NSUB
_CW = 16

_mesh = plsc.VectorSubcoreMesh(core_axis_name="c", subcore_axis_name="s")


def _make_scatter_pass(gdim, sdim):
    return functools.partial(
        pl.kernel,
        out_type=jax.ShapeDtypeStruct((_NC_CORES, _NP, _D), jnp.float32),
        mesh=_mesh,
        compiler_params=pltpu.CompilerParams(use_tc_tiling_on_sc=False),
        scratch_types=[
            pltpu.VMEM((_NCH, _CK), jnp.int32),
            pltpu.VMEM((_NCH, _CK), jnp.int32),
            pltpu.VMEM((_CK, _D), jnp.float32),
            pltpu.VMEM((_CK, _D), jnp.float32),
            pltpu.SemaphoreType.DMA,
            pltpu.SemaphoreType.DMA,
            pltpu.VMEM_SHARED((_NP, _D), jnp.float32),
        ],
    )(functools.partial(_sc_scatter_pass, gdim=gdim, sdim=sdim))


def _sc_scatter_pass(src, hidx, out, gidx_v, sidx_v, rows_a, rows_b,
                     sem_a, sem_b, acc, *, gdim, sdim):
    cid = lax.axis_index("c")
    sid = lax.axis_index("s")
    wid = sid * _NC_CORES + cid

    pltpu.async_copy(hidx.at[gdim, wid], gidx_v, sem_a)
    pltpu.async_copy(hidx.at[sdim, wid], sidx_v, sem_b)

    @pl.loop(0, _CK)
    def _zero_rows(i):
        for k in range(_D // 16):
            rows_b[i, pl.ds(k * 16, 16)] = jnp.zeros((16,), jnp.float32)

    for t in range(_RPT // _CK):
        pltpu.sync_copy(rows_b, acc.at[pl.ds(sid * _RPT + t * _CK, _CK)])
    pltpu.make_async_copy(hidx.at[gdim, wid], gidx_v, sem_a).wait()
    pltpu.make_async_copy(hidx.at[sdim, wid], sidx_v, sem_b).wait()
    plsc.subcore_barrier()

    pltpu.async_copy(src.at[gidx_v.at[0]], rows_a, sem_a)

    @pl.loop(0, _NCH - 1, step=2)
    def _chunk(j):
        pltpu.async_copy(src.at[gidx_v.at[j + 1]], rows_b, sem_b)
        pltpu.make_async_copy(src.at[gidx_v.at[j]], rows_a, sem_a).wait()
        pltpu.sync_copy(rows_a, acc.at[sidx_v.at[j]], add=True)
        pltpu.async_copy(src.at[gidx_v.at[j + 2]], rows_a, sem_a)
        pltpu.make_async_copy(src.at[gidx_v.at[j + 1]], rows_b, sem_b).wait()
        pltpu.sync_copy(rows_b, acc.at[sidx_v.at[j + 1]], add=True)

    pltpu.make_async_copy(src.at[gidx_v.at[_NCH - 1]], rows_a, sem_a).wait()
    pltpu.sync_copy(rows_a, acc.at[sidx_v.at[_NCH - 1]], add=True)

    plsc.subcore_barrier()

    nslc = _RPT // _CK
    for t in range(nslc):
        buf, sem = (rows_a, sem_a) if t % 2 == 0 else (rows_b, sem_b)
        r0 = sid * _RPT + t * _CK
        if t >= 2:
            rp = sid * _RPT + (t - 2) * _CK
            pltpu.make_async_copy(buf, out.at[cid, pl.ds(rp, _CK)],
                                  sem).wait()
        pltpu.sync_copy(acc.at[pl.ds(r0, _CK)], buf)
        pltpu.async_copy(buf, out.at[cid, pl.ds(r0, _CK)], sem)
    for t in (nslc - 2, nslc - 1):
        buf, sem = (rows_a, sem_a) if t % 2 == 0 else (rows_b, sem_b)
        r0 = sid * _RPT + t * _CK
        pltpu.make_async_copy(buf, out.at[cid, pl.ds(r0, _CK)], sem).wait()


_scatter_ne = _make_scatter_pass(0, 1)
_scatter_en = _make_scatter_pass(1, 0)


@functools.partial(
    pl.kernel,
    out_type=jax.ShapeDtypeStruct((_NC_CORES, 2, _NP, _CW), jnp.float32),
    mesh=_mesh,
    compiler_params=pltpu.CompilerParams(use_tc_tiling_on_sc=False),
    scratch_types=[
        pltpu.VMEM((_NCH, _CK), jnp.int32),
        pltpu.VMEM((_NCH, _CK), jnp.int32),
        pltpu.VMEM((_CK, _CW), jnp.float32),
        pltpu.VMEM((_RPT, _CW), jnp.float32),
        pltpu.VMEM_SHARED((_NP, _CW), jnp.float32),
        pltpu.VMEM_SHARED((_NP, _CW), jnp.float32),
    ],
)
def _sc_counts(hidx, out, nidx_v, eidx_v, ones_v, buf_v, acc_d, acc_b):
    cid = lax.axis_index("c")
    sid = lax.axis_index("s")
    wid = sid * _NC_CORES + cid

    pltpu.sync_copy(hidx.at[0, wid], nidx_v)
    pltpu.sync_copy(hidx.at[1, wid], eidx_v)

    @pl.loop(0, _CK)
    def _fill_ones(i):
        ones_v[i, :] = jnp.ones((_CW,), jnp.float32)

    @pl.loop(0, _RPT)
    def _fill_zeros(i):
        buf_v[i, :] = jnp.zeros((_CW,), jnp.float32)

    pltpu.sync_copy(buf_v, acc_d.at[pl.ds(sid * _RPT, _RPT)])
    pltpu.sync_copy(buf_v, acc_b.at[pl.ds(sid * _RPT, _RPT)])
    plsc.subcore_barrier()

    @pl.loop(0, _NCH)
    def _chunk(j):
        pltpu.sync_copy(ones_v, acc_d.at[nidx_v.at[j]], add=True)
        pltpu.sync_copy(ones_v, acc_b.at[eidx_v.at[j]], add=True)

    plsc.subcore_barrier()

    pltpu.sync_copy(acc_d.at[pl.ds(sid * _RPT, _RPT)], buf_v)
    pltpu.sync_copy(buf_v, out.at[cid, 0, pl.ds(sid * _RPT, _RPT)])
    pltpu.sync_copy(acc_b.at[pl.ds(sid * _RPT, _RPT)], buf_v)
    pltpu.sync_copy(buf_v, out.at[cid, 1, pl.ds(sid * _RPT, _RPT)])


_BLK = 1024
_GRID = _NP // _BLK


def _mm_body(x_ref, w_ref, o_ref):
    o_ref[...] = jnp.dot(x_ref[...], w_ref[...],
                         preferred_element_type=jnp.float32)


def _tc_matmul(x, w):
    return pl.pallas_call(
        _mm_body,
        grid=(_GRID,),
        in_specs=[pl.BlockSpec((_BLK, _D), lambda i: (i, 0)),
                  pl.BlockSpec((_D, _D), lambda i: (0, 0))],
        out_specs=pl.BlockSpec((_BLK, _D), lambda i: (i, 0)),
        out_shape=jax.ShapeDtypeStruct((_NP, _D), jnp.float32),
    )(x, w)


def _inv_from_counts(c_ref):
    cnt = c_ref[0, 0, :, 0:1] + c_ref[1, 0, :, 0:1]
    return jnp.where(cnt > 0.0, 1.0 / cnt, 0.0)


def _merge_b_body(p_ref, c_ref, o_ref):
    o_ref[...] = (p_ref[0] + p_ref[1]) * _inv_from_counts(c_ref)


def _tc_merge_b(parts, counts):
    return pl.pallas_call(
        _merge_b_body,
        grid=(_GRID,),
        in_specs=[
            pl.BlockSpec((_NC_CORES, _BLK, _D), lambda i: (0, i, 0)),
            pl.BlockSpec((_NC_CORES, 1, _BLK, _CW), lambda i: (0, 1, i, 0)),
        ],
        out_specs=pl.BlockSpec((_BLK, _D), lambda i: (i, 0)),
        out_shape=jax.ShapeDtypeStruct((_NP, _D), jnp.float32),
    )(parts, counts)


def _merge_d_mm_body(p_ref, c_ref, b_ref, w_ref, h_ref, o_ref):
    h = jnp.maximum(
        (p_ref[0] + p_ref[1]) * _inv_from_counts(c_ref) + b_ref[...], 0.0)
    h_ref[...] = h
    o_ref[...] = jnp.dot(h, w_ref[...], preferred_element_type=jnp.float32)


def _tc_merge_d_mm(parts, counts, bias, w):
    return pl.pallas_call(
        _merge_d_mm_body,
        grid=(_GRID,),
        in_specs=[
            pl.BlockSpec((_NC_CORES, _BLK, _D), lambda i: (0, i, 0)),
            pl.BlockSpec((_NC_CORES, 1, _BLK, _CW), lambda i: (0, 0, i, 0)),
            pl.BlockSpec((1, _D), lambda i: (0, 0)),
            pl.BlockSpec((_D, _D), lambda i: (0, 0)),
        ],
        out_specs=[pl.BlockSpec((_BLK, _D), lambda i: (i, 0)),
                   pl.BlockSpec((_BLK, _D), lambda i: (i, 0))],
        out_shape=[jax.ShapeDtypeStruct((_NP, _D), jnp.float32),
                   jax.ShapeDtypeStruct((_NP, _D), jnp.float32)],
    )(parts, counts, bias, w)


def _pool_body(p_ref, c_ref, b_ref, h0_ref, bt_ref, wl_ref, bl_ref, o_ref,
               sum_acc, max_acc, cnt_acc):
    i = pl.program_id(0)

    @pl.when(i == 0)
    def _():
        sum_acc[...] = jnp.zeros((_NG, _D), jnp.float32)
        cnt_acc[...] = jnp.zeros((_NG, _D), jnp.float32)
        max_acc[...] = jnp.full((_NG, _D), -jnp.inf, jnp.float32)

    m = (p_ref[0] + p_ref[1]) * _inv_from_counts(c_ref)
    h = jnp.maximum(h0_ref[...] + m + b_ref[...], 0.0)
    bt = bt_ref[...]
    oh = (bt == lax.broadcasted_iota(jnp.int32, (_BLK, _NG), 1)
          ).astype(jnp.float32)
    dn = (((0,), (0,)), ((), ()))
    sum_acc[...] += lax.dot_general(oh, h, dn,
                                    preferred_element_type=jnp.float32)
    cnt_acc[...] += lax.dot_general(oh, jnp.ones_like(h), dn,
                                    preferred_element_type=jnp.float32)
    g_lo = bt[0, 0]
    g_hi = bt[_BLK - 1, 0]
    for g in range(_NG):
        @pl.when((g_lo <= g) & (g <= g_hi))
        def _(g=g):
            mg = jnp.max(jnp.where(bt == g, h, -jnp.inf), axis=0,
                         keepdims=True)
            max_acc[g:g + 1, :] = jnp.maximum(max_acc[g:g + 1, :], mg)

    @pl.when(i == _GRID - 1)
    def _():
        s = sum_acc[...]
        mean = s / jnp.maximum(cnt_acc[...], 1.0)
        pooled = jnp.concatenate([mean, max_acc[...], s], axis=1)
        o_ref[...] = jnp.dot(pooled, wl_ref[...],
                             preferred_element_type=jnp.float32) + bl_ref[...]


def _tc_pool(parts, counts, bias, h0, bt, wlin, blin):
    return pl.pallas_call(
        _pool_body,
        grid=(_GRID,),
        in_specs=[pl.BlockSpec((_NC_CORES, _BLK, _D), lambda i: (0, i, 0)),
                  pl.BlockSpec((_NC_CORES, 1, _BLK, _CW),
                               lambda i: (0, 0, i, 0)),
                  pl.BlockSpec((1, _D), lambda i: (0, 0)),
                  pl.BlockSpec((_BLK, _D), lambda i: (i, 0)),
                  pl.BlockSpec((_BLK, 1), lambda i: (i, 0)),
                  pl.BlockSpec((3 * _D, _DOUT), lambda i: (0, 0)),
                  pl.BlockSpec((1, _DOUT), lambda i: (0, 0))],
        out_specs=pl.BlockSpec((_NG, _DOUT), lambda i: (0, 0)),
        out_shape=jax.ShapeDtypeStruct((_NG, _DOUT), jnp.float32),
        scratch_shapes=[pltpu.VMEM((_NG, _D), jnp.float32),
                        pltpu.VMEM((_NG, _D), jnp.float32),
                        pltpu.VMEM((_NG, _D), jnp.float32)],
    )(parts, counts, bias, h0, bt, wlin, blin)


def kernel(x, hyperedge_index, edge_attr, batch, W1, b1, W2, b2, Wlin, blin):
    del edge_attr
    hi = hyperedge_index.reshape(2, _NW, _NCH, _CK)
    xp = jnp.pad(x, ((0, _NP - _N), (0, 0)))

    def half_conv(vt, counts):
        p = _scatter_ne(vt, hi)
        m = _tc_merge_b(p, counts)
        return _scatter_en(m, hi)

    p1 = _scatter_ne(_tc_matmul(xp, W1), hi)
    counts = _sc_counts(hi)
    q1 = _scatter_en(_tc_merge_b(p1, counts), hi)
    h, xt2 = _tc_merge_d_mm(q1, counts, b1.reshape(1, _D), W2)
    q2 = half_conv(xt2, counts)

    bt = jnp.pad(batch.astype(jnp.int32), (0, _NP - _N),
                 constant_values=_NG)[:, None]
    return _tc_pool(q2, counts, b2.reshape(1, _D), h, bt, Wlin,
                    blin.reshape(1, _DOUT))
